# baseline (speedup 1.0000x reference)
import functools

import jax
import jax.numpy as jnp
from jax import lax
from jax.experimental import pallas as pl
from jax.experimental.pallas import tpu as pltpu, tpu_sc as plsc


def _build_gather(B, NP, CH):
    info = plsc.get_sparse_core_info()
    NC, NS = info.num_cores, info.num_subcores
    NW = NC * NS
    b_per_w = B // NW
    n_chunks = b_per_w // CH
    assert n_chunks % 2 == 0 and n_chunks >= 4
    mesh = plsc.VectorSubcoreMesh(core_axis_name="c", subcore_axis_name="s")

    @functools.partial(
        pl.kernel,
        mesh=mesh,
        out_type=jax.ShapeDtypeStruct((B, 128), jnp.float32),
        scratch_types=[
            pltpu.VMEM((CH,), jnp.int32),
            pltpu.VMEM((CH,), jnp.int32),
            pltpu.VMEM((CH, 128), jnp.float32),
            pltpu.VMEM((CH, 128), jnp.float32),
            pltpu.SemaphoreType.DMA((2,)),
            pltpu.SemaphoreType.DMA((2,)),
        ],
        compiler_params=pltpu.CompilerParams(use_tc_tiling_on_sc=True),
    )
    def gather(table_hbm, idx_hbm, out_hbm, idx_v0, idx_v1, rows_v0, rows_v1,
               sem_g, sem_s):
        idx_v = [idx_v0, idx_v1]
        rows_v = [rows_v0, rows_v1]
        wid = lax.axis_index("s") * NC + lax.axis_index("c")
        base = wid * b_per_w

        def load_idx(i, b):
            pltpu.sync_copy(idx_hbm.at[pl.ds(base + i * CH, CH)], idx_v[b])

        def start_gather(b):
            return pltpu.async_copy(table_hbm.at[idx_v[b]], rows_v[b],
                                    sem_g.at[b])

        def wait_gather(b):
            pltpu.make_async_copy(table_hbm.at[idx_v[b]], rows_v[b],
                                  sem_g.at[b]).wait()

        def start_store(i, b):
            return pltpu.async_copy(rows_v[b],
                                    out_hbm.at[pl.ds(base + i * CH, CH)],
                                    sem_s.at[b])

        def wait_store(b):
            pltpu.make_async_copy(rows_v[b], out_hbm.at[pl.ds(base, CH)],
                                  sem_s.at[b]).wait()

        load_idx(0, 0)
        g0 = start_gather(0)
        load_idx(1, 1)
        start_gather(1)
        g0.wait()
        start_store(0, 0)

        def body(g, carry):
            i0 = 2 * g
            i1 = 2 * g + 1
            wait_store(0)
            load_idx(i0, 0)
            start_gather(0)
            wait_gather(1)
            start_store(i1 - 2, 1)
            load_idx(i1, 1)
            wait_store(1)
            start_gather(1)
            wait_gather(0)
            start_store(i0, 0)
            return carry

        lax.fori_loop(1, n_chunks // 2, body, 0)

        wait_gather(1)
        start_store(n_chunks - 1, 1)
        wait_store(0)
        wait_store(1)

    return gather


def kernel(item_ids, table):
    batch, hist = item_ids.shape
    num_rows, dim = table.shape
    table128 = jnp.pad(table.T, ((0, 128 - dim), (0, 0))).T
    idx = item_ids.reshape(-1).astype(jnp.int32)
    out = _build_gather(idx.shape[0], table128.shape[0], 400)(table128, idx)
    return out[:, :dim].reshape(batch, hist, dim)


# speedup vs baseline: 1.0024x; 1.0024x over previous
import functools

import jax
import jax.numpy as jnp
from jax import lax
from jax.experimental import pallas as pl
from jax.experimental.pallas import tpu as pltpu, tpu_sc as plsc


def _build_gather(B, NP, CH):
    info = plsc.get_sparse_core_info()
    NC, NS = info.num_cores, info.num_subcores
    NW = NC * NS
    b_per_w = B // NW
    n_chunks = b_per_w // CH
    assert n_chunks % 2 == 0 and n_chunks >= 4
    mesh = plsc.VectorSubcoreMesh(core_axis_name="c", subcore_axis_name="s")

    @functools.partial(
        pl.kernel,
        mesh=mesh,
        out_type=jax.ShapeDtypeStruct((B, 128), jnp.float32),
        scratch_types=[
            pltpu.VMEM((CH,), jnp.int32),
            pltpu.VMEM((CH,), jnp.int32),
            pltpu.VMEM((CH, 128), jnp.float32),
            pltpu.VMEM((CH, 128), jnp.float32),
            pltpu.SemaphoreType.DMA((2,)),
            pltpu.SemaphoreType.DMA((2,)),
        ],
        compiler_params=pltpu.CompilerParams(use_tc_tiling_on_sc=True),
    )
    def gather(table_hbm, idx_hbm, out_hbm, idx_v0, idx_v1, rows_v0, rows_v1,
               sem_g, sem_s):
        idx_v = [idx_v0, idx_v1]
        rows_v = [rows_v0, rows_v1]
        wid = lax.axis_index("s") * NC + lax.axis_index("c")
        base = wid * b_per_w

        def load_idx(i, b):
            pltpu.sync_copy(idx_hbm.at[pl.ds(base + i * CH, CH)], idx_v[b])

        def start_gather(b):
            return pltpu.async_copy(table_hbm.at[idx_v[b]], rows_v[b],
                                    sem_g.at[b])

        def wait_gather(b):
            pltpu.make_async_copy(table_hbm.at[idx_v[b]], rows_v[b],
                                  sem_g.at[b]).wait()

        def start_store(i, b):
            return pltpu.async_copy(rows_v[b],
                                    out_hbm.at[pl.ds(base + i * CH, CH)],
                                    sem_s.at[b])

        def wait_store(b):
            pltpu.make_async_copy(rows_v[b], out_hbm.at[pl.ds(base, CH)],
                                  sem_s.at[b]).wait()

        load_idx(0, 0)
        g0 = start_gather(0)
        load_idx(1, 1)
        start_gather(1)
        g0.wait()
        start_store(0, 0)

        def body(g, carry):
            i0 = 2 * g
            i1 = 2 * g + 1
            wait_store(0)
            load_idx(i0, 0)
            start_gather(0)
            wait_gather(1)
            start_store(i1 - 2, 1)
            load_idx(i1, 1)
            wait_store(1)
            start_gather(1)
            wait_gather(0)
            start_store(i0, 0)
            return carry

        lax.fori_loop(1, n_chunks // 2, body, 0)

        wait_gather(1)
        start_store(n_chunks - 1, 1)
        wait_store(0)
        wait_store(1)

    return gather


def kernel(item_ids, table):
    batch, hist = item_ids.shape
    num_rows, dim = table.shape
    pad_rows = (-num_rows) % 8
    table128 = jnp.pad(table, ((0, pad_rows), (0, 128 - dim)))
    idx = item_ids.reshape(-1).astype(jnp.int32)
    out = _build_gather(idx.shape[0], table128.shape[0], 400)(table128, idx)
    return out[:, :dim].reshape(batch, hist, dim)
